# baseline (speedup 1.0000x reference)
import functools

import jax
import jax.numpy as jnp
from jax import lax
from jax.experimental import pallas as pl
from jax.experimental.pallas import tpu as pltpu
from jax.experimental.pallas import tpu_sc as plsc

B = 2048
PROX = 1024
H = 256
DIST = 256
DEN = 16
K = 32

BB = 1024
H_CHUNK = 64
CH = H_CHUNK * DIST
NEG = -3.4e38
POS = 3.4e38

NW = 32
ROWS_W = B // NW
LANE = 16


def _tc_body(x_ref, d_ref, w_ref, b_ref, a_ref, o_ref):
    prox = lax.dot_general(
        x_ref[...], w_ref[...], (((1,), (1,)), ((), ())),
        preferred_element_type=jnp.float32)
    prox = prox + b_ref[...]

    d = d_ref[...]

    def step(i, carry):
        mpos, mneg = carry
        a_chunk = a_ref[:, pl.ds(i * CH, CH)]
        v = lax.dot_general(
            d, a_chunk, (((1,), (0,)), ((), ())),
            preferred_element_type=jnp.float32)
        hi = v[:, 0:DIST]
        lo = v[:, 0:DIST]
        for j in range(1, H_CHUNK):
            s = v[:, j * DIST:(j + 1) * DIST]
            hi = jnp.maximum(hi, s)
            lo = jnp.minimum(lo, s)
        return jnp.maximum(mpos, hi), jnp.minimum(mneg, lo)

    mpos, mneg = lax.fori_loop(
        0, H // H_CHUNK, step,
        (jnp.full((BB, DIST), NEG, jnp.float32),
         jnp.full((BB, DIST), POS, jnp.float32)))

    v = jnp.where(mpos >= -mneg, mpos, mneg)
    mod = 1.0 / (1.0 + jnp.exp(-v))
    o_ref[...] = prox * mod


def _sc_topk_body(res_hbm, out_hbm, buf, obuf):
    wid = lax.axis_index("s") * 2 + lax.axis_index("c")
    rows_w = res_hbm.shape[0] // NW
    base = wid * rows_w
    pltpu.sync_copy(res_hbm.at[pl.ds(base, rows_w)], buf)

    def sortd(x):
        return plsc.sort_key_val(x, x, descending=True)[0]

    def merge16(p, q):
        qr = lax.rev(q, (0,))
        return sortd(jnp.maximum(p, qr)), sortd(jnp.minimum(p, qr))

    def merge32_keep_top(a, bq):
        ahi, alo = a
        bhi, blo = bq
        c0 = jnp.maximum(ahi, lax.rev(blo, (0,)))
        c1 = jnp.maximum(alo, lax.rev(bhi, (0,)))
        return sortd(jnp.maximum(c0, c1)), sortd(jnp.minimum(c0, c1))

    def row(r, carry):
        xs = [buf[r, pl.ds(LANE * i, LANE)] for i in range(H // LANE)]
        runs = [merge16(sortd(xs[2 * i]), sortd(xs[2 * i + 1]))
                for i in range(8)]
        while len(runs) > 1:
            runs = [merge32_keep_top(runs[2 * i], runs[2 * i + 1])
                    for i in range(len(runs) // 2)]
        _, lo = runs[0]
        thr = jnp.min(lo)
        for i in range(H // LANE):
            obuf[r, pl.ds(LANE * i, LANE)] = jnp.where(
                xs[i] >= thr, xs[i], 0.0)
        return carry

    lax.fori_loop(0, rows_w, row, 0)
    pltpu.sync_copy(obuf, out_hbm.at[pl.ds(base, rows_w)])


def _make_sc_topk(rows):
    return functools.partial(
        pl.kernel,
        mesh=plsc.VectorSubcoreMesh(core_axis_name="c", subcore_axis_name="s"),
        out_type=jax.ShapeDtypeStruct((rows, H), jnp.float32),
        compiler_params=pltpu.CompilerParams(needs_layout_passes=False),
        scratch_types=[
            pltpu.VMEM((rows // NW, H), jnp.float32),
            pltpu.VMEM((rows // NW, H), jnp.float32),
        ],
    )(_sc_topk_body)


@jax.jit
def _run(x_bf, d_bf, W_bf, b2d, A2_bf):
    def tc_chunk(xc, dc):
        return pl.pallas_call(
            _tc_body,
            grid=(1,),
            in_specs=[
                pl.BlockSpec((BB, PROX), lambda i: (i, 0)),
                pl.BlockSpec((BB, DEN), lambda i: (i, 0)),
                pl.BlockSpec((H, PROX), lambda i: (0, 0)),
                pl.BlockSpec((1, H), lambda i: (0, 0)),
                pl.BlockSpec((DEN, H * DIST), lambda i: (0, 0)),
            ],
            out_specs=pl.BlockSpec((BB, H), lambda i: (i, 0)),
            out_shape=jax.ShapeDtypeStruct((BB, H), jnp.float32),
        )(xc, dc, W_bf, b2d, A2_bf)

    sc_topk = _make_sc_topk(BB)
    ress = [tc_chunk(x_bf[i * BB:(i + 1) * BB], d_bf[i * BB:(i + 1) * BB])
            for i in range(B // BB)]
    outs = [sc_topk(r) for r in ress]
    return jnp.concatenate(outs, axis=0)


def kernel(proximal_input, distal_input, W, b, distal):
    A2 = jnp.transpose(distal, (1, 0, 2)).reshape(DEN, H * DIST)
    return _run(proximal_input.astype(jnp.bfloat16),
                distal_input.astype(jnp.bfloat16),
                W.astype(jnp.bfloat16),
                b.reshape(1, H),
                A2.astype(jnp.bfloat16))


# speedup vs baseline: 1.0413x; 1.0413x over previous
import functools

import jax
import jax.numpy as jnp
from jax import lax
from jax.experimental import pallas as pl
from jax.experimental.pallas import tpu as pltpu
from jax.experimental.pallas import tpu_sc as plsc

B = 2048
PROX = 1024
H = 256
DIST = 256
DEN = 16
K = 32

BB = 1024
H_CHUNK = 64
CH = H_CHUNK * DIST
NEG = -3.4e38
POS = 3.4e38

NW = 32
ROWS_W = B // NW
LANE = 16


def _tc_body(x_ref, d_ref, w_ref, b_ref, a_ref, o_ref):
    prox = lax.dot_general(
        x_ref[...], w_ref[...], (((1,), (1,)), ((), ())),
        preferred_element_type=jnp.float32)
    prox = prox + b_ref[...]

    d = d_ref[...]

    def step(i, carry):
        mpos, mneg = carry
        a_chunk = a_ref[:, pl.ds(i * CH, CH)]
        v = lax.dot_general(
            d, a_chunk, (((1,), (0,)), ((), ())),
            preferred_element_type=jnp.float32)
        hi = v[:, 0:DIST]
        lo = v[:, 0:DIST]
        for j in range(1, H_CHUNK):
            s = v[:, j * DIST:(j + 1) * DIST]
            hi = jnp.maximum(hi, s)
            lo = jnp.minimum(lo, s)
        return jnp.maximum(mpos, hi), jnp.minimum(mneg, lo)

    mpos, mneg = lax.fori_loop(
        0, H // H_CHUNK, step,
        (jnp.full((BB, DIST), NEG, jnp.float32),
         jnp.full((BB, DIST), POS, jnp.float32)))

    v = jnp.where(mpos >= -mneg, mpos, mneg)
    mod = 1.0 / (1.0 + jnp.exp(-v))
    o_ref[...] = prox * mod


def _sc_topk_body(res_hbm, out_hbm, buf, obuf):
    wid = lax.axis_index("s") * 2 + lax.axis_index("c")
    rows_w = res_hbm.shape[0] // NW
    base = wid * rows_w
    pltpu.sync_copy(res_hbm.at[pl.ds(base, rows_w)], buf)

    def sortd(x):
        return plsc.sort_key_val(x, x, descending=True)[0]

    def merge16(p, q):
        qr = lax.rev(q, (0,))
        return sortd(jnp.maximum(p, qr)), sortd(jnp.minimum(p, qr))

    def merge32_keep_top(a, bq):
        ahi, alo = a
        bhi, blo = bq
        c0 = jnp.maximum(ahi, lax.rev(blo, (0,)))
        c1 = jnp.maximum(alo, lax.rev(bhi, (0,)))
        return sortd(jnp.maximum(c0, c1)), sortd(jnp.minimum(c0, c1))

    def row(r, carry):
        xs = [buf[r, pl.ds(LANE * i, LANE)] for i in range(H // LANE)]
        runs = [merge16(sortd(xs[2 * i]), sortd(xs[2 * i + 1]))
                for i in range(8)]
        while len(runs) > 1:
            runs = [merge32_keep_top(runs[2 * i], runs[2 * i + 1])
                    for i in range(len(runs) // 2)]
        _, lo = runs[0]
        thr = jnp.min(lo)
        for i in range(H // LANE):
            obuf[r, pl.ds(LANE * i, LANE)] = jnp.where(
                xs[i] >= thr, xs[i], 0.0)
        return carry

    lax.fori_loop(0, rows_w, row, 0)
    pltpu.sync_copy(obuf, out_hbm.at[pl.ds(base, rows_w)])


def _make_sc_topk(rows):
    return functools.partial(
        pl.kernel,
        mesh=plsc.VectorSubcoreMesh(core_axis_name="c", subcore_axis_name="s"),
        out_type=jax.ShapeDtypeStruct((rows, H), jnp.float32),
        compiler_params=pltpu.CompilerParams(needs_layout_passes=False),
        scratch_types=[
            pltpu.VMEM((rows // NW, H), jnp.float32),
            pltpu.VMEM((rows // NW, H), jnp.float32),
        ],
    )(_sc_topk_body)


@jax.jit
def _run(x_bf, d_bf, W_bf, b2d, A2_bf):
    res = pl.pallas_call(
        _tc_body,
        grid=(B // BB,),
        in_specs=[
            pl.BlockSpec((BB, PROX), lambda i: (i, 0)),
            pl.BlockSpec((BB, DEN), lambda i: (i, 0)),
            pl.BlockSpec((H, PROX), lambda i: (0, 0)),
            pl.BlockSpec((1, H), lambda i: (0, 0)),
            pl.BlockSpec((DEN, H * DIST), lambda i: (0, 0)),
        ],
        out_specs=pl.BlockSpec((BB, H), lambda i: (i, 0)),
        out_shape=jax.ShapeDtypeStruct((B, H), jnp.float32),
    )(x_bf, d_bf, W_bf, b2d, A2_bf)
    return _make_sc_topk(B)(res)


def kernel(proximal_input, distal_input, W, b, distal):
    A2 = jnp.transpose(distal, (1, 0, 2)).reshape(DEN, H * DIST)
    return _run(proximal_input.astype(jnp.bfloat16),
                distal_input.astype(jnp.bfloat16),
                W.astype(jnp.bfloat16),
                b.reshape(1, H),
                A2.astype(jnp.bfloat16))
